# Initial kernel scaffold; baseline (speedup 1.0000x reference)
import jax
import jax.numpy as jnp
from jax.experimental import pallas as pl


def kernel(q, k, v, edge_index, Wq, Wk, Wv, Wfc, W1, as1, ad1, b1, Wl1, bl1, W2, as2, ad2, b2, Wl2, bl2, W3, as3, ad3, b3, Wl3, bl3, gamma, beta):
    raise NotImplementedError("write your pallas kernel here")


# speedup vs baseline: 107.5636x; 107.5636x over previous
import functools

import jax
import jax.numpy as jnp
from jax import lax
from jax.experimental import pallas as pl
from jax.experimental.pallas import tpu as pltpu
from jax.experimental.pallas import tpu_sc as plsc

_BS, _N, _D = 16, 512, 128
_E = 8192
_PH = 256
_DK = 64
_NSQ = _N * _N
_TILES = 16
_GPC = _BS // 2
_SLOTS_PER_TILE = _NSQ // _TILES
_EDGES_PER_TILE = _E // _TILES
_CHUNK = 128
_NCHUNK = _EDGES_PER_TILE // _CHUNK
_RB = 128
_NRB = _N // _RB


def _count_body(flat_ref, ones_ref, zeros_ref, cnt_ref,
                idx0, idx1, idx2, idx3, ones_v, shared):
    cid = lax.axis_index("c")
    sid = lax.axis_index("s")
    pltpu.sync_copy(ones_ref, ones_v)
    idx_bufs = (idx0, idx1, idx2, idx3)
    for gi in range(_GPC):
        g = cid * _GPC + gi
        pltpu.sync_copy(zeros_ref, shared.at[pl.ds(sid * _SLOTS_PER_TILE,
                                                   _SLOTS_PER_TILE)])
        plsc.subcore_barrier()
        for j in range(_NCHUNK):
            pltpu.sync_copy(flat_ref.at[g, sid, j], idx_bufs[j])
            pltpu.sync_copy(ones_v, shared.at[idx_bufs[j]], add=True)
        plsc.subcore_barrier()
        pltpu.sync_copy(shared.at[pl.ds(sid * _SLOTS_PER_TILE,
                                        _SLOTS_PER_TILE)],
                        cnt_ref.at[g, sid])
        plsc.subcore_barrier()


@functools.cache
def _get_count_kernel():
    return pl.kernel(
        _count_body,
        out_type=jax.ShapeDtypeStruct((_BS, _TILES, _SLOTS_PER_TILE),
                                      jnp.float32),
        mesh=plsc.VectorSubcoreMesh(core_axis_name="c",
                                    subcore_axis_name="s"),
        scratch_types=[
            pltpu.VMEM((_CHUNK,), jnp.int32),
            pltpu.VMEM((_CHUNK,), jnp.int32),
            pltpu.VMEM((_CHUNK,), jnp.int32),
            pltpu.VMEM((_CHUNK,), jnp.int32),
            pltpu.VMEM((_CHUNK,), jnp.float32),
            pltpu.VMEM_SHARED((_NSQ,), jnp.float32),
        ],
    )


def _lrelu(x):
    return jnp.where(x > 0, x, 0.2 * x)


def _elu(x):
    return jnp.where(x > 0, x, jnp.exp(x) - 1.0)


_PARAMS = pltpu.CompilerParams(dimension_semantics=("arbitrary", "arbitrary"))


def _full(shape):
    return pl.BlockSpec(shape, lambda g, r: (0,) * len(shape))


@functools.cache
def _make_proj(IN, F):
    def body(x_ref, W_ref, as_ref, ad_ref, xp_ref, als_ref, ald_ref):
        xb = x_ref[0]
        xp = jnp.dot(xb, W_ref[...], preferred_element_type=jnp.float32)
        xp_ref[0] = xp
        for h in range(2):
            xph = xp[:, h * F:(h + 1) * F]
            als_ref[0, h, :] = jnp.sum(xph * as_ref[h:h + 1, :], axis=1)
            ald_ref[0, h, :] = jnp.sum(xph * ad_ref[h:h + 1, :], axis=1)

    return pl.pallas_call(
        body,
        grid=(_BS, _NRB),
        in_specs=[
            pl.BlockSpec((1, _RB, IN), lambda g, r: (g, r, 0)),
            _full((IN, 2 * F)),
            _full((2, F)),
            _full((2, F)),
        ],
        out_specs=[
            pl.BlockSpec((1, _RB, 2 * F), lambda g, r: (g, r, 0)),
            pl.BlockSpec((1, 2, _RB), lambda g, r: (g, 0, r)),
            pl.BlockSpec((1, 2, _RB), lambda g, r: (g, 0, r)),
        ],
        out_shape=[
            jax.ShapeDtypeStruct((_BS, _N, 2 * F), jnp.float32),
            jax.ShapeDtypeStruct((_BS, 2, _N), jnp.float32),
            jax.ShapeDtypeStruct((_BS, 2, _N), jnp.float32),
        ],
        compiler_params=_PARAMS,
    )


@functools.cache
def _make_aggr(IN, F, concat, act_elu):
    OUT = 2 * F if concat else F

    def body(cnt_ref, xp_ref, als_ref, ald_ref, x_ref,
             b_ref, Wl_ref, bl_ref, h_ref):
        cntb = cnt_ref[0]
        mask = cntb > 0.0
        os = []
        for h in range(2):
            xph = xp_ref[0][:, h * F:(h + 1) * F]
            als_row = als_ref[0, h, :][None, :]
            ald_col = ald_ref[0, h, :][:, None]
            e = _lrelu(als_row + ald_col)
            m = jnp.max(jnp.where(mask, e, -1e30), axis=1, keepdims=True)
            p = jnp.exp(jnp.where(mask, e - m, -1e30)) * cntb
            s = jnp.sum(p, axis=1, keepdims=True)
            alpha = p / (s + 1e-16)
            os.append(jnp.dot(alpha, xph,
                              preferred_element_type=jnp.float32))
        if concat:
            o = jnp.concatenate(os, axis=1)
        else:
            o = 0.5 * (os[0] + os[1])
        lin = jnp.dot(x_ref[0], Wl_ref[...],
                      preferred_element_type=jnp.float32)
        res = o + b_ref[...] + lin + bl_ref[...]
        h_ref[0] = _elu(res) if act_elu else res

    return pl.pallas_call(
        body,
        grid=(_BS, _NRB),
        in_specs=[
            pl.BlockSpec((1, _RB, _N), lambda g, r: (g, r, 0)),
            pl.BlockSpec((1, _N, 2 * F), lambda g, r: (g, 0, 0)),
            pl.BlockSpec((1, 2, _N), lambda g, r: (g, 0, 0)),
            pl.BlockSpec((1, 2, _RB), lambda g, r: (g, 0, r)),
            pl.BlockSpec((1, _RB, IN), lambda g, r: (g, r, 0)),
            _full((1, OUT)),
            _full((IN, OUT)),
            _full((1, OUT)),
        ],
        out_specs=[pl.BlockSpec((1, _RB, OUT), lambda g, r: (g, r, 0))],
        out_shape=[jax.ShapeDtypeStruct((_BS, _N, OUT), jnp.float32)],
        compiler_params=_PARAMS,
    )


def _mha_body(q_ref, k_ref, v_ref, x3_ref, Wq_ref, Wk_ref, Wv_ref,
              Wfc_ref, gamma_ref, beta_ref, out_ref, attn_ref):
    qh = jnp.dot(q_ref[0], Wq_ref[...], preferred_element_type=jnp.float32)
    kh = jnp.dot(k_ref[0], Wk_ref[...], preferred_element_type=jnp.float32)
    vh = jnp.dot(v_ref[0], Wv_ref[...], preferred_element_type=jnp.float32)
    oh = []
    for h in range(2):
        qs = qh[:, h * _DK:(h + 1) * _DK] * (1.0 / (_DK ** 0.5))
        ks = kh[:, h * _DK:(h + 1) * _DK]
        vs = vh[:, h * _DK:(h + 1) * _DK]
        lg = lax.dot_general(qs, ks, (((1,), (1,)), ((), ())),
                             preferred_element_type=jnp.float32)
        mm = jnp.max(lg, axis=1, keepdims=True)
        ex = jnp.exp(lg - mm)
        sm = ex / jnp.sum(ex, axis=1, keepdims=True)
        attn_ref[0, h] = sm
        oh.append(jnp.dot(sm, vs, preferred_element_type=jnp.float32))
    cat = jnp.concatenate([x3_ref[0], oh[0], oh[1]], axis=1)
    out = jnp.dot(cat, Wfc_ref[...],
                  preferred_element_type=jnp.float32) + q_ref[0]
    mu = jnp.mean(out, axis=1, keepdims=True)
    var = jnp.mean((out - mu) ** 2, axis=1, keepdims=True)
    out = ((out - mu) / jnp.sqrt(var + 1e-6)) * gamma_ref[...] + beta_ref[...]
    out_ref[0] = out


_mha_call = pl.pallas_call(
    _mha_body,
    grid=(_BS, _NRB),
    in_specs=[
        pl.BlockSpec((1, _RB, _D), lambda g, r: (g, r, 0)),
        pl.BlockSpec((1, _N, _D), lambda g, r: (g, 0, 0)),
        pl.BlockSpec((1, _N, _D), lambda g, r: (g, 0, 0)),
        pl.BlockSpec((1, _RB, _D), lambda g, r: (g, r, 0)),
        _full((_D, _D)), _full((_D, _D)), _full((_D, _D)),
        _full((4 * _DK, _D)), _full((1, _D)), _full((1, _D)),
    ],
    out_specs=[
        pl.BlockSpec((1, _RB, _D), lambda g, r: (g, r, 0)),
        pl.BlockSpec((1, 2, _RB, _N), lambda g, r: (g, 0, r, 0)),
    ],
    out_shape=[
        jax.ShapeDtypeStruct((_BS, _N, _D), jnp.float32),
        jax.ShapeDtypeStruct((_BS, 2, _N, _N), jnp.float32),
    ],
    compiler_params=_PARAMS,
)


def kernel(q, k, v, edge_index, Wq, Wk, Wv, Wfc,
           W1, as1, ad1, b1, Wl1, bl1,
           W2, as2, ad2, b2, Wl2, bl2,
           W3, as3, ad3, b3, Wl3, bl3, gamma, beta):
    src = edge_index[:, 0, :].astype(jnp.int32)
    dst = edge_index[:, 1, :].astype(jnp.int32)
    flat = (dst * _N + src).reshape(_BS, _TILES, _NCHUNK, _CHUNK)
    ones = jnp.ones((_CHUNK,), jnp.float32)
    zeros = jnp.zeros((_SLOTS_PER_TILE,), jnp.float32)
    cnt = _get_count_kernel()(flat, ones, zeros).reshape(_BS, _N, _N)

    xp1, als1, ald1 = _make_proj(_D, _PH)(q, W1, as1, ad1)
    h1, = _make_aggr(_D, _PH, True, True)(
        cnt, xp1, als1, ald1, q,
        b1.reshape(1, -1), Wl1, bl1.reshape(1, -1))
    xp2, als2, ald2 = _make_proj(2 * _PH, _PH)(h1, W2, as2, ad2)
    h2, = _make_aggr(2 * _PH, _PH, True, True)(
        cnt, xp2, als2, ald2, h1,
        b2.reshape(1, -1), Wl2, bl2.reshape(1, -1))
    xp3, als3, ald3 = _make_proj(2 * _PH, 2 * _DK)(h2, W3, as3, ad3)
    x3, = _make_aggr(2 * _PH, 2 * _DK, False, False)(
        cnt, xp3, als3, ald3, h2,
        b3.reshape(1, -1), Wl3, bl3.reshape(1, -1))

    out, attn = _mha_call(q, k, v, x3, Wq, Wk, Wv, Wfc,
                          gamma.reshape(1, -1), beta.reshape(1, -1))
    return out, attn


# speedup vs baseline: 148.8455x; 1.3838x over previous
import functools

import jax
import jax.numpy as jnp
from jax import lax
from jax.experimental import pallas as pl
from jax.experimental.pallas import tpu as pltpu
from jax.experimental.pallas import tpu_sc as plsc

_BS, _N, _D = 16, 512, 128
_E = 8192
_PH = 256
_DK = 64
_NSQ = _N * _N
_TILES = 16
_GPC = _BS // 2
_SLOTS_PER_TILE = _NSQ // _TILES
_EDGES_PER_TILE = _E // _TILES
_CHUNK = 128
_NCHUNK = _EDGES_PER_TILE // _CHUNK
_RB = 256
_NRB = _N // _RB


def _count_body(flat_ref, ones_ref, zeros_ref, cnt_ref,
                idx0, idx1, idx2, idx3, ones_v, shared):
    cid = lax.axis_index("c")
    sid = lax.axis_index("s")
    pltpu.sync_copy(ones_ref, ones_v)
    idx_bufs = (idx0, idx1, idx2, idx3)
    for gi in range(_GPC):
        g = cid * _GPC + gi
        pltpu.sync_copy(zeros_ref, shared.at[pl.ds(sid * _SLOTS_PER_TILE,
                                                   _SLOTS_PER_TILE)])
        plsc.subcore_barrier()
        for j in range(_NCHUNK):
            pltpu.sync_copy(flat_ref.at[g, sid, j], idx_bufs[j])
            pltpu.sync_copy(ones_v, shared.at[idx_bufs[j]], add=True)
        plsc.subcore_barrier()
        pltpu.sync_copy(shared.at[pl.ds(sid * _SLOTS_PER_TILE,
                                        _SLOTS_PER_TILE)],
                        cnt_ref.at[g, sid])
        plsc.subcore_barrier()


@functools.cache
def _get_count_kernel():
    return pl.kernel(
        _count_body,
        out_type=jax.ShapeDtypeStruct((_BS, _TILES, _SLOTS_PER_TILE),
                                      jnp.float32),
        mesh=plsc.VectorSubcoreMesh(core_axis_name="c",
                                    subcore_axis_name="s"),
        scratch_types=[
            pltpu.VMEM((_CHUNK,), jnp.int32),
            pltpu.VMEM((_CHUNK,), jnp.int32),
            pltpu.VMEM((_CHUNK,), jnp.int32),
            pltpu.VMEM((_CHUNK,), jnp.int32),
            pltpu.VMEM((_CHUNK,), jnp.float32),
            pltpu.VMEM_SHARED((_NSQ,), jnp.float32),
        ],
    )


def _lrelu(x):
    return jnp.where(x > 0, x, 0.2 * x)


def _elu(x):
    return jnp.where(x > 0, x, jnp.exp(x) - 1.0)


_PARAMS = pltpu.CompilerParams(dimension_semantics=("parallel", "parallel"))


def _full(shape):
    return pl.BlockSpec(shape, lambda g, r: (0,) * len(shape))


@functools.cache
def _make_proj(IN, F):
    def body(x_ref, W_ref, as_ref, ad_ref, xp_ref, als_ref, ald_ref):
        xb = x_ref[0]
        xp = jnp.dot(xb, W_ref[...], preferred_element_type=jnp.float32)
        xp_ref[0] = xp
        for h in range(2):
            xph = xp[:, h * F:(h + 1) * F]
            als_ref[0, h, :] = jnp.sum(xph * as_ref[h:h + 1, :], axis=1)
            ald_ref[0, h, :] = jnp.sum(xph * ad_ref[h:h + 1, :], axis=1)

    return pl.pallas_call(
        body,
        grid=(_BS, _NRB),
        in_specs=[
            pl.BlockSpec((1, _RB, IN), lambda g, r: (g, r, 0)),
            _full((IN, 2 * F)),
            _full((2, F)),
            _full((2, F)),
        ],
        out_specs=[
            pl.BlockSpec((1, _RB, 2 * F), lambda g, r: (g, r, 0)),
            pl.BlockSpec((1, 2, _RB), lambda g, r: (g, 0, r)),
            pl.BlockSpec((1, 2, _RB), lambda g, r: (g, 0, r)),
        ],
        out_shape=[
            jax.ShapeDtypeStruct((_BS, _N, 2 * F), jnp.float32),
            jax.ShapeDtypeStruct((_BS, 2, _N), jnp.float32),
            jax.ShapeDtypeStruct((_BS, 2, _N), jnp.float32),
        ],
        compiler_params=_PARAMS,
    )


@functools.cache
def _make_aggr(IN, F, concat, act_elu):
    OUT = 2 * F if concat else F

    def body(cnt_ref, xp_ref, als_ref, ald_ref, x_ref,
             b_ref, Wl_ref, bl_ref, h_ref):
        cntb = cnt_ref[0]
        mask = cntb > 0.0
        os = []
        for h in range(2):
            xph = xp_ref[0][:, h * F:(h + 1) * F]
            als_row = als_ref[0, h, :][None, :]
            ald_col = ald_ref[0, h, :][:, None]
            e = _lrelu(als_row + ald_col)
            m = jnp.max(jnp.where(mask, e, -1e30), axis=1, keepdims=True)
            p = jnp.exp(jnp.where(mask, e - m, -1e30)) * cntb
            s = jnp.sum(p, axis=1, keepdims=True)
            alpha = p / (s + 1e-16)
            os.append(jnp.dot(alpha, xph,
                              preferred_element_type=jnp.float32))
        if concat:
            o = jnp.concatenate(os, axis=1)
        else:
            o = 0.5 * (os[0] + os[1])
        lin = jnp.dot(x_ref[0], Wl_ref[...],
                      preferred_element_type=jnp.float32)
        res = o + b_ref[...] + lin + bl_ref[...]
        h_ref[0] = _elu(res) if act_elu else res

    return pl.pallas_call(
        body,
        grid=(_BS, _NRB),
        in_specs=[
            pl.BlockSpec((1, _RB, _N), lambda g, r: (g, r, 0)),
            pl.BlockSpec((1, _N, 2 * F), lambda g, r: (g, 0, 0)),
            pl.BlockSpec((1, 2, _N), lambda g, r: (g, 0, 0)),
            pl.BlockSpec((1, 2, _RB), lambda g, r: (g, 0, r)),
            pl.BlockSpec((1, _RB, IN), lambda g, r: (g, r, 0)),
            _full((1, OUT)),
            _full((IN, OUT)),
            _full((1, OUT)),
        ],
        out_specs=[pl.BlockSpec((1, _RB, OUT), lambda g, r: (g, r, 0))],
        out_shape=[jax.ShapeDtypeStruct((_BS, _N, OUT), jnp.float32)],
        compiler_params=_PARAMS,
    )


def _mha_body(q_ref, k_ref, v_ref, x3_ref, Wq_ref, Wk_ref, Wv_ref,
              Wfc_ref, gamma_ref, beta_ref, out_ref, attn_ref):
    qh = jnp.dot(q_ref[0], Wq_ref[...], preferred_element_type=jnp.float32)
    kh = jnp.dot(k_ref[0], Wk_ref[...], preferred_element_type=jnp.float32)
    vh = jnp.dot(v_ref[0], Wv_ref[...], preferred_element_type=jnp.float32)
    oh = []
    for h in range(2):
        qs = qh[:, h * _DK:(h + 1) * _DK] * (1.0 / (_DK ** 0.5))
        ks = kh[:, h * _DK:(h + 1) * _DK]
        vs = vh[:, h * _DK:(h + 1) * _DK]
        lg = lax.dot_general(qs, ks, (((1,), (1,)), ((), ())),
                             preferred_element_type=jnp.float32)
        mm = jnp.max(lg, axis=1, keepdims=True)
        ex = jnp.exp(lg - mm)
        sm = ex / jnp.sum(ex, axis=1, keepdims=True)
        attn_ref[0, h] = sm
        oh.append(jnp.dot(sm, vs, preferred_element_type=jnp.float32))
    cat = jnp.concatenate([x3_ref[0], oh[0], oh[1]], axis=1)
    out = jnp.dot(cat, Wfc_ref[...],
                  preferred_element_type=jnp.float32) + q_ref[0]
    mu = jnp.mean(out, axis=1, keepdims=True)
    var = jnp.mean((out - mu) ** 2, axis=1, keepdims=True)
    out = ((out - mu) / jnp.sqrt(var + 1e-6)) * gamma_ref[...] + beta_ref[...]
    out_ref[0] = out


_mha_call = pl.pallas_call(
    _mha_body,
    grid=(_BS, _NRB),
    in_specs=[
        pl.BlockSpec((1, _RB, _D), lambda g, r: (g, r, 0)),
        pl.BlockSpec((1, _N, _D), lambda g, r: (g, 0, 0)),
        pl.BlockSpec((1, _N, _D), lambda g, r: (g, 0, 0)),
        pl.BlockSpec((1, _RB, _D), lambda g, r: (g, r, 0)),
        _full((_D, _D)), _full((_D, _D)), _full((_D, _D)),
        _full((4 * _DK, _D)), _full((1, _D)), _full((1, _D)),
    ],
    out_specs=[
        pl.BlockSpec((1, _RB, _D), lambda g, r: (g, r, 0)),
        pl.BlockSpec((1, 2, _RB, _N), lambda g, r: (g, 0, r, 0)),
    ],
    out_shape=[
        jax.ShapeDtypeStruct((_BS, _N, _D), jnp.float32),
        jax.ShapeDtypeStruct((_BS, 2, _N, _N), jnp.float32),
    ],
    compiler_params=_PARAMS,
)


def kernel(q, k, v, edge_index, Wq, Wk, Wv, Wfc,
           W1, as1, ad1, b1, Wl1, bl1,
           W2, as2, ad2, b2, Wl2, bl2,
           W3, as3, ad3, b3, Wl3, bl3, gamma, beta):
    src = edge_index[:, 0, :].astype(jnp.int32)
    dst = edge_index[:, 1, :].astype(jnp.int32)
    flat = (dst * _N + src).reshape(_BS, _TILES, _NCHUNK, _CHUNK)
    ones = jnp.ones((_CHUNK,), jnp.float32)
    zeros = jnp.zeros((_SLOTS_PER_TILE,), jnp.float32)
    cnt = _get_count_kernel()(flat, ones, zeros).reshape(_BS, _N, _N)

    xp1, als1, ald1 = _make_proj(_D, _PH)(q, W1, as1, ad1)
    h1, = _make_aggr(_D, _PH, True, True)(
        cnt, xp1, als1, ald1, q,
        b1.reshape(1, -1), Wl1, bl1.reshape(1, -1))
    xp2, als2, ald2 = _make_proj(2 * _PH, _PH)(h1, W2, as2, ad2)
    h2, = _make_aggr(2 * _PH, _PH, True, True)(
        cnt, xp2, als2, ald2, h1,
        b2.reshape(1, -1), Wl2, bl2.reshape(1, -1))
    xp3, als3, ald3 = _make_proj(2 * _PH, 2 * _DK)(h2, W3, as3, ad3)
    x3, = _make_aggr(2 * _PH, 2 * _DK, False, False)(
        cnt, xp3, als3, ald3, h2,
        b3.reshape(1, -1), Wl3, bl3.reshape(1, -1))

    out, attn = _mha_call(q, k, v, x3, Wq, Wk, Wv, Wfc,
                          gamma.reshape(1, -1), beta.reshape(1, -1))
    return out, attn


# speedup vs baseline: 169.8613x; 1.1412x over previous
import functools

import jax
import jax.numpy as jnp
from jax import lax
from jax.experimental import pallas as pl
from jax.experimental.pallas import tpu as pltpu
from jax.experimental.pallas import tpu_sc as plsc

_BS, _N, _D = 16, 512, 128
_E = 8192
_PH = 256
_DK = 64
_NSQ = _N * _N
_TILES = 16
_GPC = _BS // 2
_SLOTS_PER_TILE = _NSQ // _TILES
_EDGES_PER_TILE = _E // _TILES
_CHUNK = 128
_NCHUNK = _EDGES_PER_TILE // _CHUNK
_RB = 512
_NRB = _N // _RB


def _count_body(flat_ref, ones_ref, zeros_ref, cnt_ref,
                idx0, idx1, idx2, idx3, ones_v, shared):
    cid = lax.axis_index("c")
    sid = lax.axis_index("s")
    pltpu.sync_copy(ones_ref, ones_v)
    idx_bufs = (idx0, idx1, idx2, idx3)
    for gi in range(_GPC):
        g = cid * _GPC + gi
        pltpu.sync_copy(zeros_ref, shared.at[pl.ds(sid * _SLOTS_PER_TILE,
                                                   _SLOTS_PER_TILE)])
        plsc.subcore_barrier()
        for j in range(_NCHUNK):
            pltpu.sync_copy(flat_ref.at[g, sid, j], idx_bufs[j])
            pltpu.sync_copy(ones_v, shared.at[idx_bufs[j]], add=True)
        plsc.subcore_barrier()
        pltpu.sync_copy(shared.at[pl.ds(sid * _SLOTS_PER_TILE,
                                        _SLOTS_PER_TILE)],
                        cnt_ref.at[g, sid])
        plsc.subcore_barrier()


@functools.cache
def _get_count_kernel():
    return pl.kernel(
        _count_body,
        out_type=jax.ShapeDtypeStruct((_BS, _TILES, _SLOTS_PER_TILE),
                                      jnp.float32),
        mesh=plsc.VectorSubcoreMesh(core_axis_name="c",
                                    subcore_axis_name="s"),
        scratch_types=[
            pltpu.VMEM((_CHUNK,), jnp.int32),
            pltpu.VMEM((_CHUNK,), jnp.int32),
            pltpu.VMEM((_CHUNK,), jnp.int32),
            pltpu.VMEM((_CHUNK,), jnp.int32),
            pltpu.VMEM((_CHUNK,), jnp.float32),
            pltpu.VMEM_SHARED((_NSQ,), jnp.float32),
        ],
    )


def _lrelu(x):
    return jnp.where(x > 0, x, 0.2 * x)


def _elu(x):
    return jnp.where(x > 0, x, jnp.exp(x) - 1.0)


_PARAMS = pltpu.CompilerParams(dimension_semantics=("parallel", "parallel"))


def _full(shape):
    return pl.BlockSpec(shape, lambda g, r: (0,) * len(shape))


@functools.cache
def _make_proj(IN, F):
    def body(x_ref, W_ref, as_ref, ad_ref, xp_ref, als_ref, ald_ref):
        xb = x_ref[0]
        xp = jnp.dot(xb, W_ref[...], preferred_element_type=jnp.float32)
        xp_ref[0] = xp
        for h in range(2):
            xph = xp[:, h * F:(h + 1) * F]
            als_ref[0, h, :] = jnp.sum(xph * as_ref[h:h + 1, :], axis=1)
            ald_ref[0, h, :] = jnp.sum(xph * ad_ref[h:h + 1, :], axis=1)

    return pl.pallas_call(
        body,
        grid=(_BS, _NRB),
        in_specs=[
            pl.BlockSpec((1, _RB, IN), lambda g, r: (g, r, 0)),
            _full((IN, 2 * F)),
            _full((2, F)),
            _full((2, F)),
        ],
        out_specs=[
            pl.BlockSpec((1, _RB, 2 * F), lambda g, r: (g, r, 0)),
            pl.BlockSpec((1, 2, _RB), lambda g, r: (g, 0, r)),
            pl.BlockSpec((1, 2, _RB), lambda g, r: (g, 0, r)),
        ],
        out_shape=[
            jax.ShapeDtypeStruct((_BS, _N, 2 * F), jnp.float32),
            jax.ShapeDtypeStruct((_BS, 2, _N), jnp.float32),
            jax.ShapeDtypeStruct((_BS, 2, _N), jnp.float32),
        ],
        compiler_params=_PARAMS,
    )


@functools.cache
def _make_aggr(IN, F, concat, act_elu):
    OUT = 2 * F if concat else F

    def body(cnt_ref, xp_ref, als_ref, ald_ref, x_ref,
             b_ref, Wl_ref, bl_ref, h_ref):
        cntb = cnt_ref[0]
        mask = cntb > 0.0
        os = []
        for h in range(2):
            xph = xp_ref[0][:, h * F:(h + 1) * F]
            als_row = als_ref[0, h, :][None, :]
            ald_col = ald_ref[0, h, :][:, None]
            e = _lrelu(als_row + ald_col)
            m = jnp.max(jnp.where(mask, e, -1e30), axis=1, keepdims=True)
            p = jnp.exp(jnp.where(mask, e - m, -1e30)) * cntb
            s = jnp.sum(p, axis=1, keepdims=True)
            alpha = p / (s + 1e-16)
            os.append(jnp.dot(alpha, xph,
                              preferred_element_type=jnp.float32))
        if concat:
            o = jnp.concatenate(os, axis=1)
        else:
            o = 0.5 * (os[0] + os[1])
        lin = jnp.dot(x_ref[0], Wl_ref[...],
                      preferred_element_type=jnp.float32)
        res = o + b_ref[...] + lin + bl_ref[...]
        h_ref[0] = _elu(res) if act_elu else res

    return pl.pallas_call(
        body,
        grid=(_BS, _NRB),
        in_specs=[
            pl.BlockSpec((1, _RB, _N), lambda g, r: (g, r, 0)),
            pl.BlockSpec((1, _N, 2 * F), lambda g, r: (g, 0, 0)),
            pl.BlockSpec((1, 2, _N), lambda g, r: (g, 0, 0)),
            pl.BlockSpec((1, 2, _RB), lambda g, r: (g, 0, r)),
            pl.BlockSpec((1, _RB, IN), lambda g, r: (g, r, 0)),
            _full((1, OUT)),
            _full((IN, OUT)),
            _full((1, OUT)),
        ],
        out_specs=[pl.BlockSpec((1, _RB, OUT), lambda g, r: (g, r, 0))],
        out_shape=[jax.ShapeDtypeStruct((_BS, _N, OUT), jnp.float32)],
        compiler_params=_PARAMS,
    )


def _mha_body(q_ref, k_ref, v_ref, x3_ref, Wq_ref, Wk_ref, Wv_ref,
              Wfc_ref, gamma_ref, beta_ref, out_ref, attn_ref):
    qh = jnp.dot(q_ref[0], Wq_ref[...], preferred_element_type=jnp.float32)
    kh = jnp.dot(k_ref[0], Wk_ref[...], preferred_element_type=jnp.float32)
    vh = jnp.dot(v_ref[0], Wv_ref[...], preferred_element_type=jnp.float32)
    oh = []
    for h in range(2):
        qs = qh[:, h * _DK:(h + 1) * _DK] * (1.0 / (_DK ** 0.5))
        ks = kh[:, h * _DK:(h + 1) * _DK]
        vs = vh[:, h * _DK:(h + 1) * _DK]
        lg = lax.dot_general(qs, ks, (((1,), (1,)), ((), ())),
                             preferred_element_type=jnp.float32)
        mm = jnp.max(lg, axis=1, keepdims=True)
        ex = jnp.exp(lg - mm)
        sm = ex / jnp.sum(ex, axis=1, keepdims=True)
        attn_ref[0, h] = sm
        oh.append(jnp.dot(sm, vs, preferred_element_type=jnp.float32))
    cat = jnp.concatenate([x3_ref[0], oh[0], oh[1]], axis=1)
    out = jnp.dot(cat, Wfc_ref[...],
                  preferred_element_type=jnp.float32) + q_ref[0]
    mu = jnp.mean(out, axis=1, keepdims=True)
    var = jnp.mean((out - mu) ** 2, axis=1, keepdims=True)
    out = ((out - mu) / jnp.sqrt(var + 1e-6)) * gamma_ref[...] + beta_ref[...]
    out_ref[0] = out


_mha_call = pl.pallas_call(
    _mha_body,
    grid=(_BS, _NRB),
    in_specs=[
        pl.BlockSpec((1, _RB, _D), lambda g, r: (g, r, 0)),
        pl.BlockSpec((1, _N, _D), lambda g, r: (g, 0, 0)),
        pl.BlockSpec((1, _N, _D), lambda g, r: (g, 0, 0)),
        pl.BlockSpec((1, _RB, _D), lambda g, r: (g, r, 0)),
        _full((_D, _D)), _full((_D, _D)), _full((_D, _D)),
        _full((4 * _DK, _D)), _full((1, _D)), _full((1, _D)),
    ],
    out_specs=[
        pl.BlockSpec((1, _RB, _D), lambda g, r: (g, r, 0)),
        pl.BlockSpec((1, 2, _RB, _N), lambda g, r: (g, 0, r, 0)),
    ],
    out_shape=[
        jax.ShapeDtypeStruct((_BS, _N, _D), jnp.float32),
        jax.ShapeDtypeStruct((_BS, 2, _N, _N), jnp.float32),
    ],
    compiler_params=_PARAMS,
)


def kernel(q, k, v, edge_index, Wq, Wk, Wv, Wfc,
           W1, as1, ad1, b1, Wl1, bl1,
           W2, as2, ad2, b2, Wl2, bl2,
           W3, as3, ad3, b3, Wl3, bl3, gamma, beta):
    src = edge_index[:, 0, :].astype(jnp.int32)
    dst = edge_index[:, 1, :].astype(jnp.int32)
    flat = (dst * _N + src).reshape(_BS, _TILES, _NCHUNK, _CHUNK)
    ones = jnp.ones((_CHUNK,), jnp.float32)
    zeros = jnp.zeros((_SLOTS_PER_TILE,), jnp.float32)
    cnt = _get_count_kernel()(flat, ones, zeros).reshape(_BS, _N, _N)

    xp1, als1, ald1 = _make_proj(_D, _PH)(q, W1, as1, ad1)
    h1, = _make_aggr(_D, _PH, True, True)(
        cnt, xp1, als1, ald1, q,
        b1.reshape(1, -1), Wl1, bl1.reshape(1, -1))
    xp2, als2, ald2 = _make_proj(2 * _PH, _PH)(h1, W2, as2, ad2)
    h2, = _make_aggr(2 * _PH, _PH, True, True)(
        cnt, xp2, als2, ald2, h1,
        b2.reshape(1, -1), Wl2, bl2.reshape(1, -1))
    xp3, als3, ald3 = _make_proj(2 * _PH, 2 * _DK)(h2, W3, as3, ad3)
    x3, = _make_aggr(2 * _PH, 2 * _DK, False, False)(
        cnt, xp3, als3, ald3, h2,
        b3.reshape(1, -1), Wl3, bl3.reshape(1, -1))

    out, attn = _mha_call(q, k, v, x3, Wq, Wk, Wv, Wfc,
                          gamma.reshape(1, -1), beta.reshape(1, -1))
    return out, attn


# speedup vs baseline: 191.5293x; 1.1276x over previous
import functools

import jax
import jax.numpy as jnp
from jax import lax
from jax.experimental import pallas as pl
from jax.experimental.pallas import tpu as pltpu
from jax.experimental.pallas import tpu_sc as plsc

_BS, _N, _D = 16, 512, 128
_E = 8192
_PH = 256
_DK = 64
_NSQ = _N * _N
_TILES = 16
_GPC = _BS // 2
_SLOTS_PER_TILE = _NSQ // _TILES
_EDGES_PER_TILE = _E // _TILES
_CHUNK = 128
_NCHUNK = _EDGES_PER_TILE // _CHUNK
_RB = 512
_NRB = _N // _RB


_GPR = 4
_ROUNDS = _GPC // _GPR
_RCHUNK = _GPR * _NCHUNK


def _count_body(flat_ref, ones_ref, zeros_ref, cnt_ref,
                idx16, ones_v, zbuf, shared):
    cid = lax.axis_index("c")
    sid = lax.axis_index("s")
    pltpu.sync_copy(ones_ref, ones_v)
    pltpu.sync_copy(zeros_ref, zbuf)
    for r in range(_ROUNDS):
        for q in range(_GPR):
            pltpu.sync_copy(
                zbuf,
                shared.at[pl.ds(q * _NSQ + sid * _SLOTS_PER_TILE,
                                _SLOTS_PER_TILE)])
        plsc.subcore_barrier()
        pltpu.sync_copy(flat_ref.at[cid, r, sid], idx16)
        for j in range(_RCHUNK):
            pltpu.sync_copy(ones_v, shared.at[idx16.at[j]], add=True)
        plsc.subcore_barrier()
        for q in range(_GPR):
            g = cid * _GPC + r * _GPR + q
            pltpu.sync_copy(
                shared.at[pl.ds(q * _NSQ + sid * _SLOTS_PER_TILE,
                                _SLOTS_PER_TILE)],
                cnt_ref.at[g, sid])
        plsc.subcore_barrier()


@functools.cache
def _get_count_kernel():
    return pl.kernel(
        _count_body,
        out_type=jax.ShapeDtypeStruct((_BS, _TILES, _SLOTS_PER_TILE),
                                      jnp.float32),
        mesh=plsc.VectorSubcoreMesh(core_axis_name="c",
                                    subcore_axis_name="s"),
        scratch_types=[
            pltpu.VMEM((_RCHUNK, _CHUNK), jnp.int32),
            pltpu.VMEM((_CHUNK,), jnp.float32),
            pltpu.VMEM((_SLOTS_PER_TILE,), jnp.float32),
            pltpu.VMEM_SHARED((_GPR * _NSQ,), jnp.float32),
        ],
    )


def _lrelu(x):
    return jnp.where(x > 0, x, 0.2 * x)


def _elu(x):
    return jnp.where(x > 0, x, jnp.exp(x) - 1.0)


_PARAMS = pltpu.CompilerParams(dimension_semantics=("parallel", "parallel"))


def _full(shape):
    return pl.BlockSpec(shape, lambda g, r: (0,) * len(shape))


@functools.cache
def _make_proj(IN, F):
    def body(x_ref, W_ref, as_ref, ad_ref, xp_ref, als_ref, ald_ref):
        xb = x_ref[0]
        xp = jnp.dot(xb, W_ref[...], preferred_element_type=jnp.float32)
        xp_ref[0] = xp
        for h in range(2):
            xph = xp[:, h * F:(h + 1) * F]
            als_ref[0, h, :] = jnp.sum(xph * as_ref[h:h + 1, :], axis=1)
            ald_ref[0, h, :] = jnp.sum(xph * ad_ref[h:h + 1, :], axis=1)

    return pl.pallas_call(
        body,
        grid=(_BS, _NRB),
        in_specs=[
            pl.BlockSpec((1, _RB, IN), lambda g, r: (g, r, 0)),
            _full((IN, 2 * F)),
            _full((2, F)),
            _full((2, F)),
        ],
        out_specs=[
            pl.BlockSpec((1, _RB, 2 * F), lambda g, r: (g, r, 0)),
            pl.BlockSpec((1, 2, _RB), lambda g, r: (g, 0, r)),
            pl.BlockSpec((1, 2, _RB), lambda g, r: (g, 0, r)),
        ],
        out_shape=[
            jax.ShapeDtypeStruct((_BS, _N, 2 * F), jnp.float32),
            jax.ShapeDtypeStruct((_BS, 2, _N), jnp.float32),
            jax.ShapeDtypeStruct((_BS, 2, _N), jnp.float32),
        ],
        compiler_params=_PARAMS,
    )


@functools.cache
def _make_aggr(IN, F, concat, act_elu):
    OUT = 2 * F if concat else F

    def body(cnt_ref, xp_ref, als_ref, ald_ref, x_ref,
             b_ref, Wl_ref, bl_ref, h_ref):
        cntb = cnt_ref[0]
        mask = cntb > 0.0
        os = []
        for h in range(2):
            xph = xp_ref[0][:, h * F:(h + 1) * F]
            als_row = als_ref[0, h, :][None, :]
            ald_col = ald_ref[0, h, :][:, None]
            e = _lrelu(als_row + ald_col)
            m = jnp.max(jnp.where(mask, e, -1e30), axis=1, keepdims=True)
            p = jnp.exp(jnp.where(mask, e - m, -1e30)) * cntb
            s = jnp.sum(p, axis=1, keepdims=True)
            alpha = p / (s + 1e-16)
            os.append(jnp.dot(alpha, xph,
                              preferred_element_type=jnp.float32))
        if concat:
            o = jnp.concatenate(os, axis=1)
        else:
            o = 0.5 * (os[0] + os[1])
        lin = jnp.dot(x_ref[0], Wl_ref[...],
                      preferred_element_type=jnp.float32)
        res = o + b_ref[...] + lin + bl_ref[...]
        h_ref[0] = _elu(res) if act_elu else res

    return pl.pallas_call(
        body,
        grid=(_BS, _NRB),
        in_specs=[
            pl.BlockSpec((1, _RB, _N), lambda g, r: (g, r, 0)),
            pl.BlockSpec((1, _N, 2 * F), lambda g, r: (g, 0, 0)),
            pl.BlockSpec((1, 2, _N), lambda g, r: (g, 0, 0)),
            pl.BlockSpec((1, 2, _RB), lambda g, r: (g, 0, r)),
            pl.BlockSpec((1, _RB, IN), lambda g, r: (g, r, 0)),
            _full((1, OUT)),
            _full((IN, OUT)),
            _full((1, OUT)),
        ],
        out_specs=[pl.BlockSpec((1, _RB, OUT), lambda g, r: (g, r, 0))],
        out_shape=[jax.ShapeDtypeStruct((_BS, _N, OUT), jnp.float32)],
        compiler_params=_PARAMS,
    )


def _mha_body(q_ref, k_ref, v_ref, x3_ref, Wq_ref, Wk_ref, Wv_ref,
              Wfc_ref, gamma_ref, beta_ref, out_ref, attn_ref):
    qh = jnp.dot(q_ref[0], Wq_ref[...], preferred_element_type=jnp.float32)
    kh = jnp.dot(k_ref[0], Wk_ref[...], preferred_element_type=jnp.float32)
    vh = jnp.dot(v_ref[0], Wv_ref[...], preferred_element_type=jnp.float32)
    oh = []
    for h in range(2):
        qs = qh[:, h * _DK:(h + 1) * _DK] * (1.0 / (_DK ** 0.5))
        ks = kh[:, h * _DK:(h + 1) * _DK]
        vs = vh[:, h * _DK:(h + 1) * _DK]
        lg = lax.dot_general(qs, ks, (((1,), (1,)), ((), ())),
                             preferred_element_type=jnp.float32)
        mm = jnp.max(lg, axis=1, keepdims=True)
        ex = jnp.exp(lg - mm)
        sm = ex / jnp.sum(ex, axis=1, keepdims=True)
        attn_ref[0, h] = sm
        oh.append(jnp.dot(sm, vs, preferred_element_type=jnp.float32))
    cat = jnp.concatenate([x3_ref[0], oh[0], oh[1]], axis=1)
    out = jnp.dot(cat, Wfc_ref[...],
                  preferred_element_type=jnp.float32) + q_ref[0]
    mu = jnp.mean(out, axis=1, keepdims=True)
    var = jnp.mean((out - mu) ** 2, axis=1, keepdims=True)
    out = ((out - mu) / jnp.sqrt(var + 1e-6)) * gamma_ref[...] + beta_ref[...]
    out_ref[0] = out


_mha_call = pl.pallas_call(
    _mha_body,
    grid=(_BS, _NRB),
    in_specs=[
        pl.BlockSpec((1, _RB, _D), lambda g, r: (g, r, 0)),
        pl.BlockSpec((1, _N, _D), lambda g, r: (g, 0, 0)),
        pl.BlockSpec((1, _N, _D), lambda g, r: (g, 0, 0)),
        pl.BlockSpec((1, _RB, _D), lambda g, r: (g, r, 0)),
        _full((_D, _D)), _full((_D, _D)), _full((_D, _D)),
        _full((4 * _DK, _D)), _full((1, _D)), _full((1, _D)),
    ],
    out_specs=[
        pl.BlockSpec((1, _RB, _D), lambda g, r: (g, r, 0)),
        pl.BlockSpec((1, 2, _RB, _N), lambda g, r: (g, 0, r, 0)),
    ],
    out_shape=[
        jax.ShapeDtypeStruct((_BS, _N, _D), jnp.float32),
        jax.ShapeDtypeStruct((_BS, 2, _N, _N), jnp.float32),
    ],
    compiler_params=_PARAMS,
)


def kernel(q, k, v, edge_index, Wq, Wk, Wv, Wfc,
           W1, as1, ad1, b1, Wl1, bl1,
           W2, as2, ad2, b2, Wl2, bl2,
           W3, as3, ad3, b3, Wl3, bl3, gamma, beta):
    src = edge_index[:, 0, :].astype(jnp.int32)
    dst = edge_index[:, 1, :].astype(jnp.int32)
    qoff = (jnp.arange(_BS, dtype=jnp.int32) % _GPR)[:, None] * _NSQ
    flat = (dst * _N + src + qoff)
    flat = flat.reshape(2, _ROUNDS, _GPR, _TILES, _NCHUNK, _CHUNK)
    flat = flat.transpose(0, 1, 3, 2, 4, 5).reshape(
        2, _ROUNDS, _TILES, _RCHUNK, _CHUNK)
    ones = jnp.ones((_CHUNK,), jnp.float32)
    zeros = jnp.zeros((_SLOTS_PER_TILE,), jnp.float32)
    cnt = _get_count_kernel()(flat, ones, zeros).reshape(_BS, _N, _N)

    xp1, als1, ald1 = _make_proj(_D, _PH)(q, W1, as1, ad1)
    h1, = _make_aggr(_D, _PH, True, True)(
        cnt, xp1, als1, ald1, q,
        b1.reshape(1, -1), Wl1, bl1.reshape(1, -1))
    xp2, als2, ald2 = _make_proj(2 * _PH, _PH)(h1, W2, as2, ad2)
    h2, = _make_aggr(2 * _PH, _PH, True, True)(
        cnt, xp2, als2, ald2, h1,
        b2.reshape(1, -1), Wl2, bl2.reshape(1, -1))
    xp3, als3, ald3 = _make_proj(2 * _PH, 2 * _DK)(h2, W3, as3, ad3)
    x3, = _make_aggr(2 * _PH, 2 * _DK, False, False)(
        cnt, xp3, als3, ald3, h2,
        b3.reshape(1, -1), Wl3, bl3.reshape(1, -1))

    out, attn = _mha_call(q, k, v, x3, Wq, Wk, Wv, Wfc,
                          gamma.reshape(1, -1), beta.reshape(1, -1))
    return out, attn


# speedup vs baseline: 222.7086x; 1.1628x over previous
import functools

import jax
import jax.numpy as jnp
from jax import lax
from jax.experimental import pallas as pl
from jax.experimental.pallas import tpu as pltpu
from jax.experimental.pallas import tpu_sc as plsc

_BS, _N, _D = 16, 512, 128
_E = 8192
_PH = 256
_DK = 64
_NSQ = _N * _N
_TILES = 16
_GPC = _BS // 2
_SLOTS_PER_TILE = _NSQ // _TILES
_EDGES_PER_TILE = _E // _TILES
_CHUNK = 128
_NCHUNK = _EDGES_PER_TILE // _CHUNK
_RB = 512
_NRB = _N // _RB


_GPR = 4
_ROUNDS = _GPC // _GPR
_RCHUNK = _GPR * _NCHUNK


def _count_body(flat_ref, ones_ref, zeros_ref, cnt_ref,
                idx16, ones_v, zbuf, shared):
    cid = lax.axis_index("c")
    sid = lax.axis_index("s")
    pltpu.sync_copy(ones_ref, ones_v)
    pltpu.sync_copy(zeros_ref, zbuf)
    for r in range(_ROUNDS):
        for q in range(_GPR):
            pltpu.sync_copy(
                zbuf,
                shared.at[pl.ds(q * _NSQ + sid * _SLOTS_PER_TILE,
                                _SLOTS_PER_TILE)])
        plsc.subcore_barrier()
        pltpu.sync_copy(flat_ref.at[cid, r, sid], idx16)
        for j in range(_RCHUNK):
            pltpu.sync_copy(ones_v, shared.at[idx16.at[j]], add=True)
        plsc.subcore_barrier()
        for q in range(_GPR):
            g = cid * _GPC + r * _GPR + q
            pltpu.sync_copy(
                shared.at[pl.ds(q * _NSQ + sid * _SLOTS_PER_TILE,
                                _SLOTS_PER_TILE)],
                cnt_ref.at[g, sid])
        plsc.subcore_barrier()


@functools.cache
def _get_count_kernel():
    return pl.kernel(
        _count_body,
        out_type=jax.ShapeDtypeStruct((_BS, _TILES, _SLOTS_PER_TILE),
                                      jnp.float32),
        mesh=plsc.VectorSubcoreMesh(core_axis_name="c",
                                    subcore_axis_name="s"),
        scratch_types=[
            pltpu.VMEM((_RCHUNK, _CHUNK), jnp.int32),
            pltpu.VMEM((_CHUNK,), jnp.float32),
            pltpu.VMEM((_SLOTS_PER_TILE,), jnp.float32),
            pltpu.VMEM_SHARED((_GPR * _NSQ,), jnp.float32),
        ],
    )


def _lrelu(x):
    return jnp.where(x > 0, x, 0.2 * x)


def _elu(x):
    return jnp.where(x > 0, x, jnp.exp(x) - 1.0)


_PARAMS = pltpu.CompilerParams(dimension_semantics=("parallel", "parallel"))


def _full(shape):
    return pl.BlockSpec(shape, lambda g, r: (0,) * len(shape))


@functools.cache
def _make_proj(IN, F):
    def body(x_ref, W_ref, as_ref, ad_ref, xp_ref, als_ref, ald_ref):
        xb = x_ref[0]
        xp = jnp.dot(xb, W_ref[...], preferred_element_type=jnp.float32)
        xp_ref[0] = xp
        for h in range(2):
            xph = xp[:, h * F:(h + 1) * F]
            als_ref[0, h, :] = jnp.sum(xph * as_ref[h:h + 1, :], axis=1)
            ald_ref[0, h, :] = jnp.sum(xph * ad_ref[h:h + 1, :], axis=1)

    return pl.pallas_call(
        body,
        grid=(_BS, _NRB),
        in_specs=[
            pl.BlockSpec((1, _RB, IN), lambda g, r: (g, r, 0)),
            _full((IN, 2 * F)),
            _full((2, F)),
            _full((2, F)),
        ],
        out_specs=[
            pl.BlockSpec((1, _RB, 2 * F), lambda g, r: (g, r, 0)),
            pl.BlockSpec((1, 2, _RB), lambda g, r: (g, 0, r)),
            pl.BlockSpec((1, 2, _RB), lambda g, r: (g, 0, r)),
        ],
        out_shape=[
            jax.ShapeDtypeStruct((_BS, _N, 2 * F), jnp.float32),
            jax.ShapeDtypeStruct((_BS, 2, _N), jnp.float32),
            jax.ShapeDtypeStruct((_BS, 2, _N), jnp.float32),
        ],
        compiler_params=_PARAMS,
    )


def _aggr_value(cnt_ref, xp_ref, als_ref, ald_ref, x_ref,
                b_ref, Wl_ref, bl_ref, F, concat, act_elu):
    cntb = cnt_ref[0]
    mask = cntb > 0.0
    os = []
    for h in range(2):
        xph = xp_ref[0][:, h * F:(h + 1) * F]
        als_row = als_ref[0, h, :][None, :]
        ald_col = ald_ref[0, h, :][:, None]
        e = _lrelu(als_row + ald_col)
        m = jnp.max(jnp.where(mask, e, -1e30), axis=1, keepdims=True)
        p = jnp.exp(jnp.where(mask, e - m, -1e30)) * cntb
        s = jnp.sum(p, axis=1, keepdims=True)
        alpha = p / (s + 1e-16)
        os.append(jnp.dot(alpha, xph, preferred_element_type=jnp.float32))
    if concat:
        o = jnp.concatenate(os, axis=1)
    else:
        o = 0.5 * (os[0] + os[1])
    lin = jnp.dot(x_ref[0], Wl_ref[...], preferred_element_type=jnp.float32)
    res = o + b_ref[...] + lin + bl_ref[...]
    return _elu(res) if act_elu else res


def _proj_store(x, W_ref, as_ref, ad_ref, xp_ref, als_ref, ald_ref, F):
    xp = jnp.dot(x, W_ref[...], preferred_element_type=jnp.float32)
    xp_ref[0] = xp
    for h in range(2):
        xph = xp[:, h * F:(h + 1) * F]
        als_ref[0, h, :] = jnp.sum(xph * as_ref[h:h + 1, :], axis=1)
        ald_ref[0, h, :] = jnp.sum(xph * ad_ref[h:h + 1, :], axis=1)


def _aggr_specs(IN, F, OUT):
    return [
        pl.BlockSpec((1, _RB, _N), lambda g, r: (g, r, 0)),
        pl.BlockSpec((1, _N, 2 * F), lambda g, r: (g, 0, 0)),
        pl.BlockSpec((1, 2, _N), lambda g, r: (g, 0, 0)),
        pl.BlockSpec((1, 2, _RB), lambda g, r: (g, 0, r)),
        pl.BlockSpec((1, _RB, IN), lambda g, r: (g, r, 0)),
        _full((1, OUT)),
        _full((IN, OUT)),
        _full((1, OUT)),
    ]


@functools.cache
def _make_aggr_proj(IN, F, F2):
    OUT = 2 * F

    def body(cnt_ref, xp_ref, als_ref, ald_ref, x_ref,
             b_ref, Wl_ref, bl_ref, W2_ref, as2_ref, ad2_ref,
             h_ref, xp2_ref, als2_ref, ald2_ref):
        hv = _aggr_value(cnt_ref, xp_ref, als_ref, ald_ref, x_ref,
                         b_ref, Wl_ref, bl_ref, F, True, True)
        h_ref[0] = hv
        _proj_store(hv, W2_ref, as2_ref, ad2_ref,
                    xp2_ref, als2_ref, ald2_ref, F2)

    return pl.pallas_call(
        body,
        grid=(_BS, _NRB),
        in_specs=_aggr_specs(IN, F, OUT) + [
            _full((OUT, 2 * F2)), _full((2, F2)), _full((2, F2)),
        ],
        out_specs=[
            pl.BlockSpec((1, _RB, OUT), lambda g, r: (g, r, 0)),
            pl.BlockSpec((1, _RB, 2 * F2), lambda g, r: (g, r, 0)),
            pl.BlockSpec((1, 2, _RB), lambda g, r: (g, 0, r)),
            pl.BlockSpec((1, 2, _RB), lambda g, r: (g, 0, r)),
        ],
        out_shape=[
            jax.ShapeDtypeStruct((_BS, _N, OUT), jnp.float32),
            jax.ShapeDtypeStruct((_BS, _N, 2 * F2), jnp.float32),
            jax.ShapeDtypeStruct((_BS, 2, _N), jnp.float32),
            jax.ShapeDtypeStruct((_BS, 2, _N), jnp.float32),
        ],
        compiler_params=_PARAMS,
    )


@functools.cache
def _make_aggr_mha(IN, F):
    def body(cnt_ref, xp_ref, als_ref, ald_ref, x_ref,
             b_ref, Wl_ref, bl_ref,
             q_ref, k_ref, v_ref, Wq_ref, Wk_ref, Wv_ref,
             Wfc_ref, gamma_ref, beta_ref, out_ref, attn_ref):
        x3 = _aggr_value(cnt_ref, xp_ref, als_ref, ald_ref, x_ref,
                         b_ref, Wl_ref, bl_ref, F, False, False)
        qh = jnp.dot(q_ref[0], Wq_ref[...],
                     preferred_element_type=jnp.float32)
        kh = jnp.dot(k_ref[0], Wk_ref[...],
                     preferred_element_type=jnp.float32)
        vh = jnp.dot(v_ref[0], Wv_ref[...],
                     preferred_element_type=jnp.float32)
        oh = []
        for h in range(2):
            qs = qh[:, h * _DK:(h + 1) * _DK] * (1.0 / (_DK ** 0.5))
            ks = kh[:, h * _DK:(h + 1) * _DK]
            vs = vh[:, h * _DK:(h + 1) * _DK]
            lg = lax.dot_general(qs, ks, (((1,), (1,)), ((), ())),
                                 preferred_element_type=jnp.float32)
            mm = jnp.max(lg, axis=1, keepdims=True)
            ex = jnp.exp(lg - mm)
            sm = ex / jnp.sum(ex, axis=1, keepdims=True)
            attn_ref[0, h] = sm
            oh.append(jnp.dot(sm, vs, preferred_element_type=jnp.float32))
        cat = jnp.concatenate([x3, oh[0], oh[1]], axis=1)
        out = jnp.dot(cat, Wfc_ref[...],
                      preferred_element_type=jnp.float32) + q_ref[0]
        mu = jnp.mean(out, axis=1, keepdims=True)
        var = jnp.mean((out - mu) ** 2, axis=1, keepdims=True)
        out = ((out - mu) / jnp.sqrt(var + 1e-6)) * gamma_ref[...] \
            + beta_ref[...]
        out_ref[0] = out

    return pl.pallas_call(
        body,
        grid=(_BS, _NRB),
        in_specs=_aggr_specs(IN, F, F) + [
            pl.BlockSpec((1, _RB, _D), lambda g, r: (g, r, 0)),
            pl.BlockSpec((1, _N, _D), lambda g, r: (g, 0, 0)),
            pl.BlockSpec((1, _N, _D), lambda g, r: (g, 0, 0)),
            _full((_D, _D)), _full((_D, _D)), _full((_D, _D)),
            _full((4 * _DK, _D)), _full((1, _D)), _full((1, _D)),
        ],
        out_specs=[
            pl.BlockSpec((1, _RB, _D), lambda g, r: (g, r, 0)),
            pl.BlockSpec((1, 2, _RB, _N), lambda g, r: (g, 0, r, 0)),
        ],
        out_shape=[
            jax.ShapeDtypeStruct((_BS, _N, _D), jnp.float32),
            jax.ShapeDtypeStruct((_BS, 2, _N, _N), jnp.float32),
        ],
        compiler_params=_PARAMS,
    )


def kernel(q, k, v, edge_index, Wq, Wk, Wv, Wfc,
           W1, as1, ad1, b1, Wl1, bl1,
           W2, as2, ad2, b2, Wl2, bl2,
           W3, as3, ad3, b3, Wl3, bl3, gamma, beta):
    src = edge_index[:, 0, :].astype(jnp.int32)
    dst = edge_index[:, 1, :].astype(jnp.int32)
    qoff = (jnp.arange(_BS, dtype=jnp.int32) % _GPR)[:, None] * _NSQ
    flat = (dst * _N + src + qoff)
    flat = flat.reshape(2, _ROUNDS, _GPR, _TILES, _NCHUNK, _CHUNK)
    flat = flat.transpose(0, 1, 3, 2, 4, 5).reshape(
        2, _ROUNDS, _TILES, _RCHUNK, _CHUNK)
    ones = jnp.ones((_CHUNK,), jnp.float32)
    zeros = jnp.zeros((_SLOTS_PER_TILE,), jnp.float32)
    cnt = _get_count_kernel()(flat, ones, zeros).reshape(_BS, _N, _N)

    xp1, als1, ald1 = _make_proj(_D, _PH)(q, W1, as1, ad1)
    h1, xp2, als2, ald2 = _make_aggr_proj(_D, _PH, _PH)(
        cnt, xp1, als1, ald1, q,
        b1.reshape(1, -1), Wl1, bl1.reshape(1, -1), W2, as2, ad2)
    h2, xp3, als3, ald3 = _make_aggr_proj(2 * _PH, _PH, 2 * _DK)(
        cnt, xp2, als2, ald2, h1,
        b2.reshape(1, -1), Wl2, bl2.reshape(1, -1), W3, as3, ad3)
    out, attn = _make_aggr_mha(2 * _PH, 2 * _DK)(
        cnt, xp3, als3, ald3, h2,
        b3.reshape(1, -1), Wl3, bl3.reshape(1, -1),
        q, k, v, Wq, Wk, Wv, Wfc,
        gamma.reshape(1, -1), beta.reshape(1, -1))
    return out, attn


# speedup vs baseline: 228.8825x; 1.0277x over previous
import functools

import jax
import jax.numpy as jnp
from jax import lax
from jax.experimental import pallas as pl
from jax.experimental.pallas import tpu as pltpu
from jax.experimental.pallas import tpu_sc as plsc

_BS, _N, _D = 16, 512, 128
_E = 8192
_PH = 256
_DK = 64
_NSQ = _N * _N
_TILES = 16
_GPC = _BS // 2
_SLOTS_PER_TILE = _NSQ // _TILES
_EDGES_PER_TILE = _E // _TILES
_CHUNK = 128
_NCHUNK = _EDGES_PER_TILE // _CHUNK
_RB = 512
_NRB = _N // _RB


_GPR = 4
_ROUNDS = _GPC // _GPR
_RCHUNK = _GPR * _NCHUNK


def _count_body(flat_ref, ones_ref, zeros_ref, cnt_ref,
                idx16, ones_v, zbuf, shared):
    cid = lax.axis_index("c")
    sid = lax.axis_index("s")
    pltpu.sync_copy(ones_ref, ones_v)
    pltpu.sync_copy(zeros_ref, zbuf)
    for r in range(_ROUNDS):
        for q in range(_GPR):
            pltpu.sync_copy(
                zbuf,
                shared.at[pl.ds(q * _NSQ + sid * _SLOTS_PER_TILE,
                                _SLOTS_PER_TILE)])
        plsc.subcore_barrier()
        pltpu.sync_copy(flat_ref.at[cid, r, sid], idx16)
        for j in range(_RCHUNK):
            pltpu.sync_copy(ones_v, shared.at[idx16.at[j]], add=True)
        plsc.subcore_barrier()
        for q in range(_GPR):
            g = cid * _GPC + r * _GPR + q
            pltpu.sync_copy(
                shared.at[pl.ds(q * _NSQ + sid * _SLOTS_PER_TILE,
                                _SLOTS_PER_TILE)],
                cnt_ref.at[g, sid])
        plsc.subcore_barrier()


@functools.cache
def _get_count_kernel():
    return pl.kernel(
        _count_body,
        out_type=jax.ShapeDtypeStruct((_BS, _TILES, _SLOTS_PER_TILE),
                                      jnp.float32),
        mesh=plsc.VectorSubcoreMesh(core_axis_name="c",
                                    subcore_axis_name="s"),
        scratch_types=[
            pltpu.VMEM((_RCHUNK, _CHUNK), jnp.int32),
            pltpu.VMEM((_CHUNK,), jnp.float32),
            pltpu.VMEM((_SLOTS_PER_TILE,), jnp.float32),
            pltpu.VMEM_SHARED((_GPR * _NSQ,), jnp.float32),
        ],
    )


def _lrelu(x):
    return jnp.where(x > 0, x, 0.2 * x)


def _elu(x):
    return jnp.where(x > 0, x, jnp.exp(x) - 1.0)


_PARAMS = pltpu.CompilerParams(dimension_semantics=("parallel", "parallel"))


def _full(shape):
    return pl.BlockSpec(shape, lambda g, r: (0,) * len(shape))


@functools.cache
def _make_proj(IN, F):
    def body(x_ref, W_ref, as_ref, ad_ref, xp_ref, als_ref, ald_ref):
        xb = x_ref[0]
        xp = jnp.dot(xb, W_ref[...], preferred_element_type=jnp.float32)
        xp_ref[0] = xp
        for h in range(2):
            xph = xp[:, h * F:(h + 1) * F]
            als_ref[0, h, :] = jnp.sum(xph * as_ref[h:h + 1, :], axis=1)
            ald_ref[0, h, :] = jnp.sum(xph * ad_ref[h:h + 1, :], axis=1)

    return pl.pallas_call(
        body,
        grid=(_BS, _NRB),
        in_specs=[
            pl.BlockSpec((1, _RB, IN), lambda g, r: (g, r, 0)),
            _full((IN, 2 * F)),
            _full((2, F)),
            _full((2, F)),
        ],
        out_specs=[
            pl.BlockSpec((1, _RB, 2 * F), lambda g, r: (g, r, 0)),
            pl.BlockSpec((1, 2, _RB), lambda g, r: (g, 0, r)),
            pl.BlockSpec((1, 2, _RB), lambda g, r: (g, 0, r)),
        ],
        out_shape=[
            jax.ShapeDtypeStruct((_BS, _N, 2 * F), jnp.float32),
            jax.ShapeDtypeStruct((_BS, 2, _N), jnp.float32),
            jax.ShapeDtypeStruct((_BS, 2, _N), jnp.float32),
        ],
        compiler_params=_PARAMS,
    )


def _aggr_value(cnt_ref, xp_ref, als_ref, ald_ref, x_ref,
                b_ref, Wl_ref, bl_ref, F, concat, act_elu):
    cntb = cnt_ref[0]
    mask = cntb > 0.0
    os = []
    for h in range(2):
        xph = xp_ref[0][:, h * F:(h + 1) * F]
        als_row = als_ref[0, h, :][None, :]
        ald_col = ald_ref[0, h, :][:, None]
        e = _lrelu(als_row + ald_col)
        em = jnp.where(mask, e, -1e30)
        m = jnp.max(em, axis=1, keepdims=True)
        p = jnp.exp(em - m) * cntb
        s = jnp.sum(p, axis=1, keepdims=True)
        alpha = p * (1.0 / (s + 1e-16))
        os.append(jnp.dot(alpha, xph, preferred_element_type=jnp.float32))
    if concat:
        o = jnp.concatenate(os, axis=1)
    else:
        o = 0.5 * (os[0] + os[1])
    lin = jnp.dot(x_ref[0], Wl_ref[...], preferred_element_type=jnp.float32)
    res = o + b_ref[...] + lin + bl_ref[...]
    return _elu(res) if act_elu else res


def _proj_store(x, W_ref, as_ref, ad_ref, xp_ref, als_ref, ald_ref, F):
    xp = jnp.dot(x, W_ref[...], preferred_element_type=jnp.float32)
    xp_ref[0] = xp
    for h in range(2):
        xph = xp[:, h * F:(h + 1) * F]
        als_ref[0, h, :] = jnp.sum(xph * as_ref[h:h + 1, :], axis=1)
        ald_ref[0, h, :] = jnp.sum(xph * ad_ref[h:h + 1, :], axis=1)


def _aggr_specs(IN, F, OUT):
    return [
        pl.BlockSpec((1, _RB, _N), lambda g, r: (g, r, 0)),
        pl.BlockSpec((1, _N, 2 * F), lambda g, r: (g, 0, 0)),
        pl.BlockSpec((1, 2, _N), lambda g, r: (g, 0, 0)),
        pl.BlockSpec((1, 2, _RB), lambda g, r: (g, 0, r)),
        pl.BlockSpec((1, _RB, IN), lambda g, r: (g, r, 0)),
        _full((1, OUT)),
        _full((IN, OUT)),
        _full((1, OUT)),
    ]


@functools.cache
def _make_aggr_proj(IN, F, F2):
    OUT = 2 * F

    def body(cnt_ref, xp_ref, als_ref, ald_ref, x_ref,
             b_ref, Wl_ref, bl_ref, W2_ref, as2_ref, ad2_ref,
             h_ref, xp2_ref, als2_ref, ald2_ref):
        hv = _aggr_value(cnt_ref, xp_ref, als_ref, ald_ref, x_ref,
                         b_ref, Wl_ref, bl_ref, F, True, True)
        h_ref[0] = hv
        _proj_store(hv, W2_ref, as2_ref, ad2_ref,
                    xp2_ref, als2_ref, ald2_ref, F2)

    return pl.pallas_call(
        body,
        grid=(_BS, _NRB),
        in_specs=_aggr_specs(IN, F, OUT) + [
            _full((OUT, 2 * F2)), _full((2, F2)), _full((2, F2)),
        ],
        out_specs=[
            pl.BlockSpec((1, _RB, OUT), lambda g, r: (g, r, 0)),
            pl.BlockSpec((1, _RB, 2 * F2), lambda g, r: (g, r, 0)),
            pl.BlockSpec((1, 2, _RB), lambda g, r: (g, 0, r)),
            pl.BlockSpec((1, 2, _RB), lambda g, r: (g, 0, r)),
        ],
        out_shape=[
            jax.ShapeDtypeStruct((_BS, _N, OUT), jnp.float32),
            jax.ShapeDtypeStruct((_BS, _N, 2 * F2), jnp.float32),
            jax.ShapeDtypeStruct((_BS, 2, _N), jnp.float32),
            jax.ShapeDtypeStruct((_BS, 2, _N), jnp.float32),
        ],
        compiler_params=_PARAMS,
    )


@functools.cache
def _make_aggr_mha(IN, F):
    def body(cnt_ref, xp_ref, als_ref, ald_ref, x_ref,
             b_ref, Wl_ref, bl_ref,
             q_ref, k_ref, v_ref, Wq_ref, Wk_ref, Wv_ref,
             Wfc_ref, gamma_ref, beta_ref, out_ref, attn_ref):
        x3 = _aggr_value(cnt_ref, xp_ref, als_ref, ald_ref, x_ref,
                         b_ref, Wl_ref, bl_ref, F, False, False)
        qh = jnp.dot(q_ref[0], Wq_ref[...],
                     preferred_element_type=jnp.float32)
        kh = jnp.dot(k_ref[0], Wk_ref[...],
                     preferred_element_type=jnp.float32)
        vh = jnp.dot(v_ref[0], Wv_ref[...],
                     preferred_element_type=jnp.float32)
        oh = []
        for h in range(2):
            qs = qh[:, h * _DK:(h + 1) * _DK] * (1.0 / (_DK ** 0.5))
            ks = kh[:, h * _DK:(h + 1) * _DK]
            vs = vh[:, h * _DK:(h + 1) * _DK]
            lg = lax.dot_general(qs, ks, (((1,), (1,)), ((), ())),
                                 preferred_element_type=jnp.float32)
            mm = jnp.max(lg, axis=1, keepdims=True)
            ex = jnp.exp(lg - mm)
            sm = ex * (1.0 / jnp.sum(ex, axis=1, keepdims=True))
            attn_ref[0, h] = sm
            oh.append(jnp.dot(sm, vs, preferred_element_type=jnp.float32))
        cat = jnp.concatenate([x3, oh[0], oh[1]], axis=1)
        out = jnp.dot(cat, Wfc_ref[...],
                      preferred_element_type=jnp.float32) + q_ref[0]
        mu = jnp.mean(out, axis=1, keepdims=True)
        var = jnp.mean((out - mu) ** 2, axis=1, keepdims=True)
        out = ((out - mu) / jnp.sqrt(var + 1e-6)) * gamma_ref[...] \
            + beta_ref[...]
        out_ref[0] = out

    return pl.pallas_call(
        body,
        grid=(_BS, _NRB),
        in_specs=_aggr_specs(IN, F, F) + [
            pl.BlockSpec((1, _RB, _D), lambda g, r: (g, r, 0)),
            pl.BlockSpec((1, _N, _D), lambda g, r: (g, 0, 0)),
            pl.BlockSpec((1, _N, _D), lambda g, r: (g, 0, 0)),
            _full((_D, _D)), _full((_D, _D)), _full((_D, _D)),
            _full((4 * _DK, _D)), _full((1, _D)), _full((1, _D)),
        ],
        out_specs=[
            pl.BlockSpec((1, _RB, _D), lambda g, r: (g, r, 0)),
            pl.BlockSpec((1, 2, _RB, _N), lambda g, r: (g, 0, r, 0)),
        ],
        out_shape=[
            jax.ShapeDtypeStruct((_BS, _N, _D), jnp.float32),
            jax.ShapeDtypeStruct((_BS, 2, _N, _N), jnp.float32),
        ],
        compiler_params=_PARAMS,
    )


def kernel(q, k, v, edge_index, Wq, Wk, Wv, Wfc,
           W1, as1, ad1, b1, Wl1, bl1,
           W2, as2, ad2, b2, Wl2, bl2,
           W3, as3, ad3, b3, Wl3, bl3, gamma, beta):
    src = edge_index[:, 0, :].astype(jnp.int32)
    dst = edge_index[:, 1, :].astype(jnp.int32)
    qoff = (jnp.arange(_BS, dtype=jnp.int32) % _GPR)[:, None] * _NSQ
    flat = (dst * _N + src + qoff)
    flat = flat.reshape(2, _ROUNDS, _GPR, _TILES, _NCHUNK, _CHUNK)
    flat = flat.transpose(0, 1, 3, 2, 4, 5).reshape(
        2, _ROUNDS, _TILES, _RCHUNK, _CHUNK)
    ones = jnp.ones((_CHUNK,), jnp.float32)
    zeros = jnp.zeros((_SLOTS_PER_TILE,), jnp.float32)
    cnt = _get_count_kernel()(flat, ones, zeros).reshape(_BS, _N, _N)

    xp1, als1, ald1 = _make_proj(_D, _PH)(q, W1, as1, ad1)
    h1, xp2, als2, ald2 = _make_aggr_proj(_D, _PH, _PH)(
        cnt, xp1, als1, ald1, q,
        b1.reshape(1, -1), Wl1, bl1.reshape(1, -1), W2, as2, ad2)
    h2, xp3, als3, ald3 = _make_aggr_proj(2 * _PH, _PH, 2 * _DK)(
        cnt, xp2, als2, ald2, h1,
        b2.reshape(1, -1), Wl2, bl2.reshape(1, -1), W3, as3, ad3)
    out, attn = _make_aggr_mha(2 * _PH, 2 * _DK)(
        cnt, xp3, als3, ald3, h2,
        b3.reshape(1, -1), Wl3, bl3.reshape(1, -1),
        q, k, v, Wq, Wk, Wv, Wfc,
        gamma.reshape(1, -1), beta.reshape(1, -1))
    return out, attn
